# baseline (speedup 1.0000x reference)
import functools

import jax
import jax.numpy as jnp
from jax import lax
from jax.experimental import pallas as pl
from jax.experimental.pallas import tpu as pltpu
from jax.experimental.pallas import tpu_sc as plsc

_NBUF = 2
_CHUNK = 256
_UNROLL = 8


@functools.cache
def _make_gather(B, V, D):
    info = plsc.get_sparse_core_info()
    NC, NS = info.num_cores, info.num_subcores
    NW = NC * NS
    assert B % NW == 0 and D == 32
    per_w = B // NW
    C = _CHUNK
    assert per_w % (C * _NBUF) == 0
    n_groups = per_w // (C * _NBUF)
    assert n_groups >= 2
    H = D // 2
    assert V % NS == 0
    v_per_w = V // NS

    mesh = plsc.VectorSubcoreMesh(core_axis_name="c", subcore_axis_name="s")

    @functools.partial(
        pl.kernel,
        mesh=mesh,
        out_type=jax.ShapeDtypeStruct((B, D), jnp.float32),
        scratch_types=(
            [pltpu.VMEM((_NBUF, C), jnp.int32),
             pltpu.VMEM((_NBUF, C, D), jnp.bfloat16),
             pltpu.VMEM((_NBUF, C, D), jnp.float32),
             pltpu.VMEM_SHARED((V, D), jnp.bfloat16)]
            + [pltpu.SemaphoreType.DMA] * (3 * _NBUF)
        ),
        compiler_params=pltpu.CompilerParams(
            use_tc_tiling_on_sc=False, needs_layout_passes=False),
    )
    def k(table_hbm, idx_hbm, out_hbm, idx_v, rows_bf, rows_f, tab_sh,
          *sems):
        sem_idx = sems[:_NBUF]
        sem_g = sems[_NBUF:2 * _NBUF]
        sem_out = sems[2 * _NBUF:]
        cid = lax.axis_index("c")
        sid = lax.axis_index("s")
        wid = sid * NC + cid
        base = wid * per_w

        def idx_copy(j, b):
            return pltpu.make_async_copy(
                idx_hbm.at[pl.ds(base + j * C, C)], idx_v.at[b], sem_idx[b])

        def gather_copy(b):
            return pltpu.make_async_copy(
                tab_sh.at[idx_v.at[b]], rows_bf.at[b], sem_g[b])

        def out_copy(j, b):
            return pltpu.make_async_copy(
                rows_f.at[b], out_hbm.at[pl.ds(base + j * C, C)], sem_out[b])

        def widen(b):
            def body(i, carry):
                r0 = i * _UNROLL
                for u in range(_UNROLL):
                    r = r0 + u
                    w = plsc.bitcast(rows_bf[b, r, :], jnp.int32)
                    lo = plsc.bitcast(w << 16, jnp.float32)
                    hi = plsc.bitcast(w & jnp.int32(-65536), jnp.float32)
                    rows_f[b, r, pl.ds(0, H)] = lo
                    rows_f[b, r, pl.ds(H, H)] = hi
                return carry

            lax.fori_loop(0, C // _UNROLL, body, 0)

        for b in range(_NBUF):
            idx_copy(b, b).start()

        v0 = sid * v_per_w
        pltpu.sync_copy(table_hbm.at[pl.ds(v0, v_per_w)],
                        tab_sh.at[pl.ds(v0, v_per_w)])
        plsc.subcore_barrier()

        for b in range(_NBUF):
            idx_copy(b, b).wait()
            gather_copy(b).start()
        for b in range(_NBUF):
            gather_copy(b).wait()
            widen(b)
            out_copy(b, b).start()
            idx_copy(_NBUF + b, b).start()

        def group(g, carry):
            j0 = g * _NBUF
            for b in range(_NBUF):
                out_copy(j0 - _NBUF + b, b).wait()
                idx_copy(j0 + b, b).wait()
                gather_copy(b).start()
            for b in range(_NBUF):
                gather_copy(b).wait()
                widen(b)
                out_copy(j0 + b, b).start()
                idx_copy(j0 + _NBUF + b, b).start()
            return carry

        lax.fori_loop(1, n_groups - 1, group, 0)

        j0 = (n_groups - 1) * _NBUF
        for b in range(_NBUF):
            out_copy(j0 - _NBUF + b, b).wait()
            idx_copy(j0 + b, b).wait()
            gather_copy(b).start()
        for b in range(_NBUF):
            gather_copy(b).wait()
            widen(b)
            out_copy(j0 + b, b).start()
        for b in range(_NBUF):
            out_copy(j0 + b, b).wait()

    return k


def kernel(x, table):
    B = x.shape[0] * x.shape[1]
    V, D = table.shape
    idx = x.reshape(B).astype(jnp.int32)
    tb = (table.astype(jnp.bfloat16)
          .reshape(V, 2, D // 2).transpose(0, 2, 1).reshape(V, D))
    out = _make_gather(B, V, D)(tb, idx)
    return out.reshape(x.shape + (D,))


# speedup vs baseline: 1.1692x; 1.1692x over previous
import functools

import jax
import jax.numpy as jnp
from jax import lax
from jax.experimental import pallas as pl
from jax.experimental.pallas import tpu as pltpu
from jax.experimental.pallas import tpu_sc as plsc

_NBUF = 4
_ROWS = 1


@functools.cache
def _make_gather(N, T, D):
    info = plsc.get_sparse_core_info()
    NC, NS = info.num_cores, info.num_subcores
    NW = NC * NS
    assert N % NW == 0
    rows_w = N // NW
    R = _ROWS
    assert rows_w % (R * _NBUF) == 0
    n_groups = rows_w // (R * _NBUF)
    assert n_groups >= 2

    mesh = plsc.VectorSubcoreMesh(core_axis_name="c", subcore_axis_name="s")

    @functools.partial(
        pl.kernel,
        mesh=mesh,
        out_type=jax.ShapeDtypeStruct((N, T, D), jnp.float32),
        scratch_types=(
            [pltpu.VMEM((_NBUF, T), jnp.int32),
             pltpu.VMEM((_NBUF, T, D), jnp.float32)]
            + [pltpu.SemaphoreType.DMA] * (3 * _NBUF)
        ),
        compiler_params=pltpu.CompilerParams(use_tc_tiling_on_sc=False),
    )
    def k(table_hbm, idx_hbm, out_hbm, idx_v, rows_v, *sems):
        sem_idx = sems[:_NBUF]
        sem_g = sems[_NBUF:2 * _NBUF]
        sem_out = sems[2 * _NBUF:]
        wid = lax.axis_index("s") * NC + lax.axis_index("c")
        base = wid * rows_w

        def idx_copy(j, b):
            return pltpu.make_async_copy(
                idx_hbm.at[base + j], idx_v.at[b], sem_idx[b])

        def gather_copy(b):
            return pltpu.make_async_copy(
                table_hbm.at[idx_v.at[b]], rows_v.at[b], sem_g[b])

        def out_copy(j, b):
            return pltpu.make_async_copy(
                rows_v.at[b], out_hbm.at[base + j], sem_out[b])

        for b in range(_NBUF):
            idx_copy(b, b).start()

        for b in range(_NBUF):
            idx_copy(b, b).wait()
            gather_copy(b).start()
        for b in range(_NBUF):
            gather_copy(b).wait()
            out_copy(b, b).start()
            idx_copy(_NBUF + b, b).start()

        def group(g, carry):
            j0 = g * _NBUF
            for b in range(_NBUF):
                out_copy(j0 - _NBUF + b, b).wait()
                idx_copy(j0 + b, b).wait()
                gather_copy(b).start()
            for b in range(_NBUF):
                gather_copy(b).wait()
                out_copy(j0 + b, b).start()
                idx_copy(j0 + _NBUF + b, b).start()
            return carry

        lax.fori_loop(1, n_groups - 1, group, 0)

        j0 = (n_groups - 1) * _NBUF
        for b in range(_NBUF):
            out_copy(j0 - _NBUF + b, b).wait()
            idx_copy(j0 + b, b).wait()
            gather_copy(b).start()
        for b in range(_NBUF):
            gather_copy(b).wait()
            out_copy(j0 + b, b).start()
        for b in range(_NBUF):
            out_copy(j0 + b, b).wait()

    return k


def kernel(x, table):
    N, T = x.shape
    D = table.shape[1]
    return _make_gather(N, T, D)(table, x.astype(jnp.int32))
